# Initial kernel scaffold; baseline (speedup 1.0000x reference)
import jax
import jax.numpy as jnp
from jax.experimental import pallas as pl


def kernel(x_macro, x_pico, x_femto, ei_mm, ei_pp, ei_ff, ei_mp, ei_mf, W1l_mm, W1r_mm, b1_mm, W2l_mm, W2r_mm, b2_mm, W1l_pp, W1r_pp, b1_pp, W2l_pp, W2r_pp, b2_pp, W1l_ff, W1r_ff, b1_ff, W2l_ff, W2r_ff, b2_ff, W1l_mp, W1r_mp, b1_mp, W2l_mp, W2r_mp, b2_mp, W1l_mf, W1r_mf, b1_mf, W2l_mf, W2r_mf, b2_mf, ln_g, ln_b, projW_macro, projb_macro, projW_pico, projb_pico, projW_femto, projb_femto):
    raise NotImplementedError("write your pallas kernel here")


# speedup vs baseline: 5.4709x; 5.4709x over previous
import functools

import jax
import jax.numpy as jnp
from jax import lax
from jax.experimental import pallas as pl
from jax.experimental.pallas import tpu as pltpu
from jax.experimental.pallas import tpu_sc as plsc

N = 10000
E = 160000
IN_DIM = 128
HID = 256
OUT = 256

NSC = 2
NTILE = 16
CHUNK = 128
N_PAD = 10240
ROWS_PER_TILE = N_PAD // NTILE
NBUF = 2


def _wid_axes():
    c = lax.axis_index("c")
    s = lax.axis_index("s")
    return c, s


def _segsum1_body(x, src, dst, zrow, z1d, ones_hbm, part, cnt_out,
                  idx_s, idx_d, rows, ones_v, acc, cacc, sem0, sem1):
    c, s = _wid_axes()
    sems = (sem0, sem1)
    base_row = s * ROWS_PER_TILE
    pltpu.sync_copy(zrow.at[pl.ds(base_row, ROWS_PER_TILE)],
                    acc.at[pl.ds(base_row, ROWS_PER_TILE)])
    pltpu.sync_copy(z1d.at[pl.ds(base_row, ROWS_PER_TILE)],
                    cacc.at[pl.ds(base_row, ROWS_PER_TILE)])
    pltpu.sync_copy(ones_hbm, ones_v)
    plsc.subcore_barrier()

    e_per_sc = E // NSC
    n_chunks = e_per_sc // CHUNK
    n_iter = (n_chunks + NTILE - 1) // NTILE
    n_groups = (n_iter + NBUF - 1) // NBUF

    for b in range(NBUF):
        off = c * e_per_sc + (s + b * NTILE) * CHUNK
        pltpu.sync_copy(src.at[pl.ds(off, CHUNK)], idx_s.at[b])
        pltpu.sync_copy(dst.at[pl.ds(off, CHUNK)], idx_d.at[b])
        pltpu.async_copy(x.at[idx_s.at[b]], rows.at[b], sems[b])

    def group(g, _):
        for b in range(NBUF):
            j = s + (g * NBUF + b) * NTILE

            @pl.when(j < n_chunks)
            def _():
                pltpu.make_async_copy(x.at[pl.ds(0, CHUNK)], rows.at[b],
                                      sems[b]).wait()
                jn = j + NBUF * NTILE

                @pl.when(jn < n_chunks)
                def _():
                    offn = c * e_per_sc + jn * CHUNK
                    pltpu.sync_copy(rows.at[b], acc.at[idx_d.at[b]],
                                    add=True)
                    pltpu.sync_copy(ones_v, cacc.at[idx_d.at[b]], add=True)
                    pltpu.sync_copy(src.at[pl.ds(offn, CHUNK)], idx_s.at[b])
                    pltpu.sync_copy(dst.at[pl.ds(offn, CHUNK)], idx_d.at[b])
                    pltpu.async_copy(x.at[idx_s.at[b]], rows.at[b], sems[b])

                @pl.when(jn >= n_chunks)
                def _():
                    pltpu.sync_copy(rows.at[b], acc.at[idx_d.at[b]],
                                    add=True)
                    pltpu.sync_copy(ones_v, cacc.at[idx_d.at[b]], add=True)
        return None

    lax.fori_loop(0, n_groups, group, None)
    plsc.subcore_barrier()

    pltpu.sync_copy(acc.at[pl.ds(base_row, ROWS_PER_TILE)],
                    part.at[c, pl.ds(base_row, ROWS_PER_TILE)])
    pltpu.sync_copy(cacc.at[pl.ds(base_row, ROWS_PER_TILE)],
                    cnt_out.at[c, pl.ds(base_row, ROWS_PER_TILE)])


_segsum1 = pl.kernel(
    _segsum1_body,
    out_type=[
        jax.ShapeDtypeStruct((NSC, N_PAD, 128), jnp.float32),
        jax.ShapeDtypeStruct((NSC, N_PAD), jnp.float32),
    ],
    mesh=plsc.VectorSubcoreMesh(core_axis_name="c", subcore_axis_name="s"),
    scratch_types=[
        pltpu.VMEM((NBUF, CHUNK), jnp.int32),
        pltpu.VMEM((NBUF, CHUNK), jnp.int32),
        pltpu.VMEM((NBUF, CHUNK, 128), jnp.float32),
        pltpu.VMEM((CHUNK,), jnp.float32),
        pltpu.VMEM_SHARED((N_PAD, 128), jnp.float32),
        pltpu.VMEM_SHARED((N_PAD,), jnp.float32),
        pltpu.SemaphoreType.DMA,
        pltpu.SemaphoreType.DMA,
    ],
)


def _segsum2_body(x_lo, x_hi, src, dst, zrow, out_lo, out_hi,
                  idx_s, idx_d, rows, acc, sem0, sem1):
    c, s = _wid_axes()
    sems = (sem0, sem1)
    base_row = s * ROWS_PER_TILE
    pltpu.sync_copy(zrow.at[pl.ds(base_row, ROWS_PER_TILE)],
                    acc.at[pl.ds(base_row, ROWS_PER_TILE)])
    plsc.subcore_barrier()

    n_chunks = E // CHUNK
    n_iter = (n_chunks + NTILE - 1) // NTILE
    n_groups = (n_iter + NBUF - 1) // NBUF

    def start_gather(b):
        @pl.when(c == 0)
        def _():
            pltpu.async_copy(x_lo.at[idx_s.at[b]], rows.at[b], sems[b])

        @pl.when(c == 1)
        def _():
            pltpu.async_copy(x_hi.at[idx_s.at[b]], rows.at[b], sems[b])

    for b in range(NBUF):
        off = (s + b * NTILE) * CHUNK
        pltpu.sync_copy(src.at[pl.ds(off, CHUNK)], idx_s.at[b])
        pltpu.sync_copy(dst.at[pl.ds(off, CHUNK)], idx_d.at[b])
        start_gather(b)

    def group(g, _):
        for b in range(NBUF):
            j = s + (g * NBUF + b) * NTILE

            @pl.when(j < n_chunks)
            def _():
                pltpu.make_async_copy(x_lo.at[pl.ds(0, CHUNK)], rows.at[b],
                                      sems[b]).wait()
                jn = j + NBUF * NTILE

                @pl.when(jn < n_chunks)
                def _():
                    offn = jn * CHUNK
                    pltpu.sync_copy(rows.at[b], acc.at[idx_d.at[b]],
                                    add=True)
                    pltpu.sync_copy(src.at[pl.ds(offn, CHUNK)], idx_s.at[b])
                    pltpu.sync_copy(dst.at[pl.ds(offn, CHUNK)], idx_d.at[b])
                    start_gather(b)

                @pl.when(jn >= n_chunks)
                def _():
                    pltpu.sync_copy(rows.at[b], acc.at[idx_d.at[b]],
                                    add=True)
        return None

    lax.fori_loop(0, n_groups, group, None)
    plsc.subcore_barrier()

    @pl.when(c == 0)
    def _():
        pltpu.sync_copy(acc.at[pl.ds(base_row, ROWS_PER_TILE)],
                        out_lo.at[pl.ds(base_row, ROWS_PER_TILE)])

    @pl.when(c == 1)
    def _():
        pltpu.sync_copy(acc.at[pl.ds(base_row, ROWS_PER_TILE)],
                        out_hi.at[pl.ds(base_row, ROWS_PER_TILE)])


_segsum2 = pl.kernel(
    _segsum2_body,
    out_type=[
        jax.ShapeDtypeStruct((N_PAD, 128), jnp.float32),
        jax.ShapeDtypeStruct((N_PAD, 128), jnp.float32),
    ],
    mesh=plsc.VectorSubcoreMesh(core_axis_name="c", subcore_axis_name="s"),
    scratch_types=[
        pltpu.VMEM((NBUF, CHUNK), jnp.int32),
        pltpu.VMEM((NBUF, CHUNK), jnp.int32),
        pltpu.VMEM((NBUF, CHUNK, 128), jnp.float32),
        pltpu.VMEM_SHARED((N_PAD, 128), jnp.float32),
        pltpu.SemaphoreType.DMA,
        pltpu.SemaphoreType.DMA,
    ],
)


BLK = 1024


def _mean(part_ref, cpart_ref):
    agg = part_ref[0] + part_ref[1]
    cnt = cpart_ref[0] + cpart_ref[1]
    return agg * (1.0 / jnp.clip(cnt, 1.0, None))[:, None]


def _dense1_one_body(part, cpart, x, Wl, Wr, b, out_lo, out_hi):
    mean = _mean(part, cpart)
    h = (jnp.dot(mean, Wl[...].T, preferred_element_type=jnp.float32)
         + jnp.dot(x[...], Wr[...].T, preferred_element_type=jnp.float32)
         + b[...])
    h = jnp.maximum(h, 0.0)
    out_lo[...] = h[:, :128]
    out_hi[...] = h[:, 128:]


def _dense1_two_body(part_a, cpart_a, part_b, cpart_b, x,
                     Wl_a, Wl_b, Wr_a, Wr_b, b_a, b_b, out_lo, out_hi):
    mean_a = _mean(part_a, cpart_a)
    mean_b = _mean(part_b, cpart_b)
    h = (jnp.dot(mean_a, Wl_a[...].T, preferred_element_type=jnp.float32)
         + jnp.dot(mean_b, Wl_b[...].T, preferred_element_type=jnp.float32)
         + jnp.dot(x[...], (Wr_a[...] + Wr_b[...]).T,
                   preferred_element_type=jnp.float32)
         + b_a[...] + b_b[...])
    h = jnp.maximum(h, 0.0)
    out_lo[...] = h[:, :128]
    out_hi[...] = h[:, 128:]


def _ln_relu_proj(v, g, b, pw, pb):
    mu = jnp.mean(v, axis=-1, keepdims=True)
    var = jnp.mean((v - mu) * (v - mu), axis=-1, keepdims=True)
    h = (v - mu) * lax.rsqrt(var + 1e-5) * g + b
    h = jnp.maximum(h, 0.0)
    return jnp.dot(h, pw[...].T, preferred_element_type=jnp.float32) + pb[...]


def _dense2_one_body(agg_lo, agg_hi, cpart, x_lo, x_hi, Wl, Wr, b,
                     g, lb, pw, pb, out):
    agg = jnp.concatenate([agg_lo[...], agg_hi[...]], axis=1)
    cnt = cpart[0] + cpart[1]
    mean = agg * (1.0 / jnp.clip(cnt, 1.0, None))[:, None]
    x = jnp.concatenate([x_lo[...], x_hi[...]], axis=1)
    v = (jnp.dot(mean, Wl[...].T, preferred_element_type=jnp.float32)
         + jnp.dot(x, Wr[...].T, preferred_element_type=jnp.float32)
         + b[...])
    out[...] = _ln_relu_proj(v, g[...], lb[...], pw, pb)


def _dense2_two_body(agg_a_lo, agg_a_hi, cpart_a, agg_b_lo, agg_b_hi,
                     cpart_b, x_lo, x_hi, Wl_a, Wl_b, Wr_a, Wr_b, b_a, b_b,
                     g, lb, pw, pb, out):
    agg_a = jnp.concatenate([agg_a_lo[...], agg_a_hi[...]], axis=1)
    cnt_a = cpart_a[0] + cpart_a[1]
    mean_a = agg_a * (1.0 / jnp.clip(cnt_a, 1.0, None))[:, None]
    agg_b = jnp.concatenate([agg_b_lo[...], agg_b_hi[...]], axis=1)
    cnt_b = cpart_b[0] + cpart_b[1]
    mean_b = agg_b * (1.0 / jnp.clip(cnt_b, 1.0, None))[:, None]
    x = jnp.concatenate([x_lo[...], x_hi[...]], axis=1)
    v = (jnp.dot(mean_a, Wl_a[...].T, preferred_element_type=jnp.float32)
         + jnp.dot(mean_b, Wl_b[...].T, preferred_element_type=jnp.float32)
         + jnp.dot(x, (Wr_a[...] + Wr_b[...]).T,
                   preferred_element_type=jnp.float32)
         + b_a[...] + b_b[...])
    out[...] = _ln_relu_proj(v, g[...], lb[...], pw, pb)


def _row_block(i):
    return (i, 0)


def _part_block(i):
    return (0, i, 0)


def _full_block(*_):
    return (0, 0)


_spec_part128 = pl.BlockSpec((NSC, BLK, 128), _part_block)
_spec_cnt = pl.BlockSpec((NSC, BLK), lambda i: (0, i))
_spec_row128 = pl.BlockSpec((BLK, 128), _row_block)
_spec_row256 = pl.BlockSpec((BLK, 256), _row_block)
_spec_w128 = pl.BlockSpec((256, 128), _full_block)
_spec_w256 = pl.BlockSpec((256, 256), _full_block)
_spec_vec = pl.BlockSpec((1, 256), _full_block)

_dense1_one = pl.pallas_call(
    _dense1_one_body,
    grid=(N_PAD // BLK,),
    in_specs=[_spec_part128, _spec_cnt, _spec_row128,
              _spec_w128, _spec_w128, _spec_vec],
    out_specs=[_spec_row128, _spec_row128],
    out_shape=[jax.ShapeDtypeStruct((N_PAD, 128), jnp.float32)] * 2,
)

_dense1_two = pl.pallas_call(
    _dense1_two_body,
    grid=(N_PAD // BLK,),
    in_specs=[_spec_part128, _spec_cnt, _spec_part128, _spec_cnt,
              _spec_row128, _spec_w128, _spec_w128, _spec_w128, _spec_w128,
              _spec_vec, _spec_vec],
    out_specs=[_spec_row128, _spec_row128],
    out_shape=[jax.ShapeDtypeStruct((N_PAD, 128), jnp.float32)] * 2,
)

_dense2_one = pl.pallas_call(
    _dense2_one_body,
    grid=(N_PAD // BLK,),
    in_specs=[_spec_row128, _spec_row128, _spec_cnt,
              _spec_row128, _spec_row128,
              _spec_w256, _spec_w256, _spec_vec,
              _spec_vec, _spec_vec, _spec_w256, _spec_vec],
    out_specs=_spec_row256,
    out_shape=jax.ShapeDtypeStruct((N_PAD, 256), jnp.float32),
)

_dense2_two = pl.pallas_call(
    _dense2_two_body,
    grid=(N_PAD // BLK,),
    in_specs=[_spec_row128, _spec_row128, _spec_cnt,
              _spec_row128, _spec_row128, _spec_cnt,
              _spec_row128, _spec_row128,
              _spec_w256, _spec_w256, _spec_w256, _spec_w256,
              _spec_vec, _spec_vec,
              _spec_vec, _spec_vec, _spec_w256, _spec_vec],
    out_specs=_spec_row256,
    out_shape=jax.ShapeDtypeStruct((N_PAD, 256), jnp.float32),
)


@jax.jit
def kernel(x_macro, x_pico, x_femto,
           ei_mm, ei_pp, ei_ff, ei_mp, ei_mf,
           W1l_mm, W1r_mm, b1_mm, W2l_mm, W2r_mm, b2_mm,
           W1l_pp, W1r_pp, b1_pp, W2l_pp, W2r_pp, b2_pp,
           W1l_ff, W1r_ff, b1_ff, W2l_ff, W2r_ff, b2_ff,
           W1l_mp, W1r_mp, b1_mp, W2l_mp, W2r_mp, b2_mp,
           W1l_mf, W1r_mf, b1_mf, W2l_mf, W2r_mf, b2_mf,
           ln_g, ln_b,
           projW_macro, projb_macro,
           projW_pico, projb_pico,
           projW_femto, projb_femto):
    zrow = jnp.zeros((N_PAD, 128), jnp.float32)
    z1d = jnp.zeros((N_PAD,), jnp.float32)
    ones1d = jnp.ones((CHUNK,), jnp.float32)
    pad = ((0, N_PAD - N), (0, 0))
    x_macro = jnp.pad(x_macro, pad)
    x_pico = jnp.pad(x_pico, pad)
    x_femto = jnp.pad(x_femto, pad)

    srcs = {"mm": ei_mm[0], "pp": ei_pp[0], "ff": ei_ff[0],
            "mp": ei_mp[0], "mf": ei_mf[0]}
    dsts = {"mm": ei_mm[1], "pp": ei_pp[1], "ff": ei_ff[1],
            "mp": ei_mp[1], "mf": ei_mf[1]}
    xsrc1 = {"mm": x_macro, "pp": x_pico, "ff": x_femto,
             "mp": x_macro, "mf": x_macro}

    part, cpart = {}, {}
    for r in ("mm", "pp", "ff", "mp", "mf"):
        part[r], cpart[r] = _segsum1(xsrc1[r], srcs[r], dsts[r],
                                     zrow, z1d, ones1d)

    b1 = {k: v.reshape(1, -1) for k, v in
          {"mm": b1_mm, "pp": b1_pp, "ff": b1_ff,
           "mp": b1_mp, "mf": b1_mf}.items()}

    m1_lo, m1_hi = _dense1_one(part["mm"], cpart["mm"], x_macro,
                               W1l_mm, W1r_mm, b1["mm"])
    p1_lo, p1_hi = _dense1_two(part["pp"], cpart["pp"],
                               part["mp"], cpart["mp"], x_pico,
                               W1l_pp, W1l_mp, W1r_pp, W1r_mp,
                               b1["pp"], b1["mp"])
    f1_lo, f1_hi = _dense1_two(part["ff"], cpart["ff"],
                               part["mf"], cpart["mf"], x_femto,
                               W1l_ff, W1l_mf, W1r_ff, W1r_mf,
                               b1["ff"], b1["mf"])

    xsrc2 = {"mm": (m1_lo, m1_hi), "pp": (p1_lo, p1_hi),
             "ff": (f1_lo, f1_hi), "mp": (m1_lo, m1_hi),
             "mf": (m1_lo, m1_hi)}
    agg2 = {}
    for r in ("mm", "pp", "ff", "mp", "mf"):
        lo, hi = xsrc2[r]
        agg2[r] = _segsum2(lo, hi, srcs[r], dsts[r], zrow)

    g = ln_g.reshape(1, -1)
    lb = ln_b.reshape(1, -1)
    out_m = _dense2_one(agg2["mm"][0], agg2["mm"][1], cpart["mm"],
                        m1_lo, m1_hi, W2l_mm, W2r_mm,
                        b2_mm.reshape(1, -1), g, lb,
                        projW_macro, projb_macro.reshape(1, -1))
    out_p = _dense2_two(agg2["pp"][0], agg2["pp"][1], cpart["pp"],
                        agg2["mp"][0], agg2["mp"][1], cpart["mp"],
                        p1_lo, p1_hi, W2l_pp, W2l_mp, W2r_pp, W2r_mp,
                        b2_pp.reshape(1, -1), b2_mp.reshape(1, -1), g, lb,
                        projW_pico, projb_pico.reshape(1, -1))
    out_f = _dense2_two(agg2["ff"][0], agg2["ff"][1], cpart["ff"],
                        agg2["mf"][0], agg2["mf"][1], cpart["mf"],
                        f1_lo, f1_hi, W2l_ff, W2l_mf, W2r_ff, W2r_mf,
                        b2_ff.reshape(1, -1), b2_mf.reshape(1, -1), g, lb,
                        projW_femto, projb_femto.reshape(1, -1))
    return (out_m[:N], out_p[:N], out_f[:N])


# speedup vs baseline: 7.2931x; 1.3331x over previous
import functools

import jax
import jax.numpy as jnp
from jax import lax
from jax.experimental import pallas as pl
from jax.experimental.pallas import tpu as pltpu
from jax.experimental.pallas import tpu_sc as plsc

N = 10000
E = 160000
IN_DIM = 128
HID = 256
OUT = 256

NSC = 2
NTILE = 16
CHUNK = 128
N_PAD = 10240
ROWS_PER_TILE = N_PAD // NTILE
NBUF = 2


def _wid_axes():
    c = lax.axis_index("c")
    s = lax.axis_index("s")
    return c, s


def _segsum1_body(x, eip, zrow, z1d, ones_hbm, part, cnt_out,
                  idx, rows, ones_v, acc, cacc,
                  semg0, semg1, semi0, semi1):
    c, s = _wid_axes()
    semg = (semg0, semg1)
    semi = (semi0, semi1)
    base_row = s * ROWS_PER_TILE
    pltpu.sync_copy(zrow.at[pl.ds(base_row, ROWS_PER_TILE)],
                    acc.at[pl.ds(base_row, ROWS_PER_TILE)])
    pltpu.sync_copy(z1d.at[pl.ds(base_row, ROWS_PER_TILE)],
                    cacc.at[pl.ds(base_row, ROWS_PER_TILE)])
    pltpu.sync_copy(ones_hbm, ones_v)
    plsc.subcore_barrier()

    e_per_sc = E // NSC
    n_chunks = e_per_sc // CHUNK
    n_iter = (n_chunks + NTILE - 1) // NTILE
    n_groups = (n_iter + NBUF - 1) // NBUF
    cbase = c * n_chunks

    for b in range(NBUF):
        pltpu.sync_copy(eip.at[cbase + s + b * NTILE], idx.at[b, 0])
        pltpu.async_copy(x.at[idx.at[b, 0, 0]], rows.at[b], semg[b])

    def group(gp, _):
        for phase in range(2):
            pn = 1 - phase
            for b in range(NBUF):
                j = s + ((gp * 2 + phase) * NBUF + b) * NTILE

                @pl.when(j < n_chunks)
                def _(b=b, phase=phase, pn=pn, j=j):
                    pltpu.make_async_copy(x.at[pl.ds(0, CHUNK)],
                                          rows.at[b], semg[b]).wait()
                    jn = j + NBUF * NTILE

                    @pl.when(jn < n_chunks)
                    def _():
                        pltpu.async_copy(eip.at[cbase + jn], idx.at[b, pn],
                                         semi[b])

                    pltpu.sync_copy(rows.at[b],
                                    acc.at[idx.at[b, phase, 1]], add=True)
                    pltpu.sync_copy(ones_v,
                                    cacc.at[idx.at[b, phase, 1]], add=True)

                    @pl.when(jn < n_chunks)
                    def _():
                        pltpu.make_async_copy(eip.at[0], idx.at[b, pn],
                                              semi[b]).wait()
                        pltpu.async_copy(x.at[idx.at[b, pn, 0]],
                                         rows.at[b], semg[b])
        return None

    lax.fori_loop(0, n_groups // 2, group, None)
    plsc.subcore_barrier()

    pltpu.sync_copy(acc.at[pl.ds(base_row, ROWS_PER_TILE)],
                    part.at[c, pl.ds(base_row, ROWS_PER_TILE)])
    pltpu.sync_copy(cacc.at[pl.ds(base_row, ROWS_PER_TILE)],
                    cnt_out.at[c, pl.ds(base_row, ROWS_PER_TILE)])


_segsum1 = pl.kernel(
    _segsum1_body,
    out_type=[
        jax.ShapeDtypeStruct((NSC, N_PAD, 128), jnp.float32),
        jax.ShapeDtypeStruct((NSC, N_PAD), jnp.float32),
    ],
    mesh=plsc.VectorSubcoreMesh(core_axis_name="c", subcore_axis_name="s"),
    scratch_types=[
        pltpu.VMEM((NBUF, 2, 2, CHUNK), jnp.int32),
        pltpu.VMEM((NBUF, CHUNK, 128), jnp.float32),
        pltpu.VMEM((CHUNK,), jnp.float32),
        pltpu.VMEM_SHARED((N_PAD, 128), jnp.float32),
        pltpu.VMEM_SHARED((N_PAD,), jnp.float32),
        pltpu.SemaphoreType.DMA,
        pltpu.SemaphoreType.DMA,
        pltpu.SemaphoreType.DMA,
        pltpu.SemaphoreType.DMA,
    ],
)


def _segsum2_body(x_lo, x_hi, eip, zrow, out_lo, out_hi,
                  idx, rows, acc, semg0, semg1, semi0, semi1):
    c, s = _wid_axes()
    semg = (semg0, semg1)
    semi = (semi0, semi1)
    base_row = s * ROWS_PER_TILE
    pltpu.sync_copy(zrow.at[pl.ds(base_row, ROWS_PER_TILE)],
                    acc.at[pl.ds(base_row, ROWS_PER_TILE)])
    plsc.subcore_barrier()

    n_chunks = E // CHUNK
    n_iter = (n_chunks + NTILE - 1) // NTILE
    n_groups = (n_iter + NBUF - 1) // NBUF

    def start_gather(b, p):
        @pl.when(c == 0)
        def _():
            pltpu.async_copy(x_lo.at[idx.at[b, p, 0]], rows.at[b], semg[b])

        @pl.when(c == 1)
        def _():
            pltpu.async_copy(x_hi.at[idx.at[b, p, 0]], rows.at[b], semg[b])

    for b in range(NBUF):
        pltpu.sync_copy(eip.at[s + b * NTILE], idx.at[b, 0])
        start_gather(b, 0)

    def group(gp, _):
        for phase in range(2):
            pn = 1 - phase
            for b in range(NBUF):
                j = s + ((gp * 2 + phase) * NBUF + b) * NTILE

                @pl.when(j < n_chunks)
                def _(b=b, phase=phase, pn=pn, j=j):
                    pltpu.make_async_copy(x_lo.at[pl.ds(0, CHUNK)],
                                          rows.at[b], semg[b]).wait()
                    jn = j + NBUF * NTILE

                    @pl.when(jn < n_chunks)
                    def _():
                        pltpu.async_copy(eip.at[jn], idx.at[b, pn], semi[b])

                    pltpu.sync_copy(rows.at[b],
                                    acc.at[idx.at[b, phase, 1]], add=True)

                    @pl.when(jn < n_chunks)
                    def _():
                        pltpu.make_async_copy(eip.at[0], idx.at[b, pn],
                                              semi[b]).wait()
                        start_gather(b, pn)
        return None

    lax.fori_loop(0, n_groups // 2, group, None)
    plsc.subcore_barrier()

    @pl.when(c == 0)
    def _():
        pltpu.sync_copy(acc.at[pl.ds(base_row, ROWS_PER_TILE)],
                        out_lo.at[pl.ds(base_row, ROWS_PER_TILE)])

    @pl.when(c == 1)
    def _():
        pltpu.sync_copy(acc.at[pl.ds(base_row, ROWS_PER_TILE)],
                        out_hi.at[pl.ds(base_row, ROWS_PER_TILE)])


_segsum2 = pl.kernel(
    _segsum2_body,
    out_type=[
        jax.ShapeDtypeStruct((N_PAD, 128), jnp.float32),
        jax.ShapeDtypeStruct((N_PAD, 128), jnp.float32),
    ],
    mesh=plsc.VectorSubcoreMesh(core_axis_name="c", subcore_axis_name="s"),
    scratch_types=[
        pltpu.VMEM((NBUF, 2, 2, CHUNK), jnp.int32),
        pltpu.VMEM((NBUF, CHUNK, 128), jnp.float32),
        pltpu.VMEM_SHARED((N_PAD, 128), jnp.float32),
        pltpu.SemaphoreType.DMA,
        pltpu.SemaphoreType.DMA,
        pltpu.SemaphoreType.DMA,
        pltpu.SemaphoreType.DMA,
    ],
)


BLK = 1024


def _mean(part_ref, cpart_ref):
    agg = part_ref[0] + part_ref[1]
    cnt = cpart_ref[0] + cpart_ref[1]
    return agg * (1.0 / jnp.clip(cnt, 1.0, None))[:, None]


def _dense1_one_body(part, cpart, x, Wl, Wr, b, out_lo, out_hi):
    mean = _mean(part, cpart)
    h = (jnp.dot(mean, Wl[...].T, preferred_element_type=jnp.float32)
         + jnp.dot(x[...], Wr[...].T, preferred_element_type=jnp.float32)
         + b[...])
    h = jnp.maximum(h, 0.0)
    out_lo[...] = h[:, :128]
    out_hi[...] = h[:, 128:]


def _dense1_two_body(part_a, cpart_a, part_b, cpart_b, x,
                     Wl_a, Wl_b, Wr_a, Wr_b, b_a, b_b, out_lo, out_hi):
    mean_a = _mean(part_a, cpart_a)
    mean_b = _mean(part_b, cpart_b)
    h = (jnp.dot(mean_a, Wl_a[...].T, preferred_element_type=jnp.float32)
         + jnp.dot(mean_b, Wl_b[...].T, preferred_element_type=jnp.float32)
         + jnp.dot(x[...], (Wr_a[...] + Wr_b[...]).T,
                   preferred_element_type=jnp.float32)
         + b_a[...] + b_b[...])
    h = jnp.maximum(h, 0.0)
    out_lo[...] = h[:, :128]
    out_hi[...] = h[:, 128:]


def _ln_relu_proj(v, g, b, pw, pb):
    mu = jnp.mean(v, axis=-1, keepdims=True)
    var = jnp.mean((v - mu) * (v - mu), axis=-1, keepdims=True)
    h = (v - mu) * lax.rsqrt(var + 1e-5) * g + b
    h = jnp.maximum(h, 0.0)
    return jnp.dot(h, pw[...].T, preferred_element_type=jnp.float32) + pb[...]


def _dense2_one_body(agg_lo, agg_hi, cpart, x_lo, x_hi, Wl, Wr, b,
                     g, lb, pw, pb, out):
    agg = jnp.concatenate([agg_lo[...], agg_hi[...]], axis=1)
    cnt = cpart[0] + cpart[1]
    mean = agg * (1.0 / jnp.clip(cnt, 1.0, None))[:, None]
    x = jnp.concatenate([x_lo[...], x_hi[...]], axis=1)
    v = (jnp.dot(mean, Wl[...].T, preferred_element_type=jnp.float32)
         + jnp.dot(x, Wr[...].T, preferred_element_type=jnp.float32)
         + b[...])
    out[...] = _ln_relu_proj(v, g[...], lb[...], pw, pb)


def _dense2_two_body(agg_a_lo, agg_a_hi, cpart_a, agg_b_lo, agg_b_hi,
                     cpart_b, x_lo, x_hi, Wl_a, Wl_b, Wr_a, Wr_b, b_a, b_b,
                     g, lb, pw, pb, out):
    agg_a = jnp.concatenate([agg_a_lo[...], agg_a_hi[...]], axis=1)
    cnt_a = cpart_a[0] + cpart_a[1]
    mean_a = agg_a * (1.0 / jnp.clip(cnt_a, 1.0, None))[:, None]
    agg_b = jnp.concatenate([agg_b_lo[...], agg_b_hi[...]], axis=1)
    cnt_b = cpart_b[0] + cpart_b[1]
    mean_b = agg_b * (1.0 / jnp.clip(cnt_b, 1.0, None))[:, None]
    x = jnp.concatenate([x_lo[...], x_hi[...]], axis=1)
    v = (jnp.dot(mean_a, Wl_a[...].T, preferred_element_type=jnp.float32)
         + jnp.dot(mean_b, Wl_b[...].T, preferred_element_type=jnp.float32)
         + jnp.dot(x, (Wr_a[...] + Wr_b[...]).T,
                   preferred_element_type=jnp.float32)
         + b_a[...] + b_b[...])
    out[...] = _ln_relu_proj(v, g[...], lb[...], pw, pb)


def _row_block(i):
    return (i, 0)


def _part_block(i):
    return (0, i, 0)


def _full_block(*_):
    return (0, 0)


_spec_part128 = pl.BlockSpec((NSC, BLK, 128), _part_block)
_spec_cnt = pl.BlockSpec((NSC, BLK), lambda i: (0, i))
_spec_row128 = pl.BlockSpec((BLK, 128), _row_block)
_spec_row256 = pl.BlockSpec((BLK, 256), _row_block)
_spec_w128 = pl.BlockSpec((256, 128), _full_block)
_spec_w256 = pl.BlockSpec((256, 256), _full_block)
_spec_vec = pl.BlockSpec((1, 256), _full_block)

_dense1_one = pl.pallas_call(
    _dense1_one_body,
    grid=(N_PAD // BLK,),
    in_specs=[_spec_part128, _spec_cnt, _spec_row128,
              _spec_w128, _spec_w128, _spec_vec],
    out_specs=[_spec_row128, _spec_row128],
    out_shape=[jax.ShapeDtypeStruct((N_PAD, 128), jnp.float32)] * 2,
)

_dense1_two = pl.pallas_call(
    _dense1_two_body,
    grid=(N_PAD // BLK,),
    in_specs=[_spec_part128, _spec_cnt, _spec_part128, _spec_cnt,
              _spec_row128, _spec_w128, _spec_w128, _spec_w128, _spec_w128,
              _spec_vec, _spec_vec],
    out_specs=[_spec_row128, _spec_row128],
    out_shape=[jax.ShapeDtypeStruct((N_PAD, 128), jnp.float32)] * 2,
)

_dense2_one = pl.pallas_call(
    _dense2_one_body,
    grid=(N_PAD // BLK,),
    in_specs=[_spec_row128, _spec_row128, _spec_cnt,
              _spec_row128, _spec_row128,
              _spec_w256, _spec_w256, _spec_vec,
              _spec_vec, _spec_vec, _spec_w256, _spec_vec],
    out_specs=_spec_row256,
    out_shape=jax.ShapeDtypeStruct((N_PAD, 256), jnp.float32),
)

_dense2_two = pl.pallas_call(
    _dense2_two_body,
    grid=(N_PAD // BLK,),
    in_specs=[_spec_row128, _spec_row128, _spec_cnt,
              _spec_row128, _spec_row128, _spec_cnt,
              _spec_row128, _spec_row128,
              _spec_w256, _spec_w256, _spec_w256, _spec_w256,
              _spec_vec, _spec_vec,
              _spec_vec, _spec_vec, _spec_w256, _spec_vec],
    out_specs=_spec_row256,
    out_shape=jax.ShapeDtypeStruct((N_PAD, 256), jnp.float32),
)


@jax.jit
def kernel(x_macro, x_pico, x_femto,
           ei_mm, ei_pp, ei_ff, ei_mp, ei_mf,
           W1l_mm, W1r_mm, b1_mm, W2l_mm, W2r_mm, b2_mm,
           W1l_pp, W1r_pp, b1_pp, W2l_pp, W2r_pp, b2_pp,
           W1l_ff, W1r_ff, b1_ff, W2l_ff, W2r_ff, b2_ff,
           W1l_mp, W1r_mp, b1_mp, W2l_mp, W2r_mp, b2_mp,
           W1l_mf, W1r_mf, b1_mf, W2l_mf, W2r_mf, b2_mf,
           ln_g, ln_b,
           projW_macro, projb_macro,
           projW_pico, projb_pico,
           projW_femto, projb_femto):
    zrow = jnp.zeros((N_PAD, 128), jnp.float32)
    z1d = jnp.zeros((N_PAD,), jnp.float32)
    ones1d = jnp.ones((CHUNK,), jnp.float32)
    pad = ((0, N_PAD - N), (0, 0))
    x_macro = jnp.pad(x_macro, pad)
    x_pico = jnp.pad(x_pico, pad)
    x_femto = jnp.pad(x_femto, pad)

    def _pack(ei):
        return jnp.stack([ei[0].reshape(-1, CHUNK),
                          ei[1].reshape(-1, CHUNK)], axis=1)

    eips = {"mm": _pack(ei_mm), "pp": _pack(ei_pp), "ff": _pack(ei_ff),
            "mp": _pack(ei_mp), "mf": _pack(ei_mf)}
    xsrc1 = {"mm": x_macro, "pp": x_pico, "ff": x_femto,
             "mp": x_macro, "mf": x_macro}

    part, cpart = {}, {}
    for r in ("mm", "pp", "ff", "mp", "mf"):
        part[r], cpart[r] = _segsum1(xsrc1[r], eips[r], zrow, z1d, ones1d)

    b1 = {k: v.reshape(1, -1) for k, v in
          {"mm": b1_mm, "pp": b1_pp, "ff": b1_ff,
           "mp": b1_mp, "mf": b1_mf}.items()}

    m1_lo, m1_hi = _dense1_one(part["mm"], cpart["mm"], x_macro,
                               W1l_mm, W1r_mm, b1["mm"])
    p1_lo, p1_hi = _dense1_two(part["pp"], cpart["pp"],
                               part["mp"], cpart["mp"], x_pico,
                               W1l_pp, W1l_mp, W1r_pp, W1r_mp,
                               b1["pp"], b1["mp"])
    f1_lo, f1_hi = _dense1_two(part["ff"], cpart["ff"],
                               part["mf"], cpart["mf"], x_femto,
                               W1l_ff, W1l_mf, W1r_ff, W1r_mf,
                               b1["ff"], b1["mf"])

    xsrc2 = {"mm": (m1_lo, m1_hi), "pp": (p1_lo, p1_hi),
             "ff": (f1_lo, f1_hi), "mp": (m1_lo, m1_hi),
             "mf": (m1_lo, m1_hi)}
    agg2 = {}
    for r in ("mm", "pp", "ff", "mp", "mf"):
        lo, hi = xsrc2[r]
        agg2[r] = _segsum2(lo, hi, eips[r], zrow)

    g = ln_g.reshape(1, -1)
    lb = ln_b.reshape(1, -1)
    out_m = _dense2_one(agg2["mm"][0], agg2["mm"][1], cpart["mm"],
                        m1_lo, m1_hi, W2l_mm, W2r_mm,
                        b2_mm.reshape(1, -1), g, lb,
                        projW_macro, projb_macro.reshape(1, -1))
    out_p = _dense2_two(agg2["pp"][0], agg2["pp"][1], cpart["pp"],
                        agg2["mp"][0], agg2["mp"][1], cpart["mp"],
                        p1_lo, p1_hi, W2l_pp, W2l_mp, W2r_pp, W2r_mp,
                        b2_pp.reshape(1, -1), b2_mp.reshape(1, -1), g, lb,
                        projW_pico, projb_pico.reshape(1, -1))
    out_f = _dense2_two(agg2["ff"][0], agg2["ff"][1], cpart["ff"],
                        agg2["mf"][0], agg2["mf"][1], cpart["mf"],
                        f1_lo, f1_hi, W2l_ff, W2l_mf, W2r_ff, W2r_mf,
                        b2_ff.reshape(1, -1), b2_mf.reshape(1, -1), g, lb,
                        projW_femto, projb_femto.reshape(1, -1))
    return (out_m[:N], out_p[:N], out_f[:N])
